# Initial kernel scaffold; baseline (speedup 1.0000x reference)
import jax
import jax.numpy as jnp
from jax.experimental import pallas as pl


def kernel(g, in_feat, W1_rel, W1_root, b1, W2_rel, W2_root, b2):
    raise NotImplementedError("write your pallas kernel here")


# speedup vs baseline: 2.7468x; 2.7468x over previous
import functools

import jax
import jax.numpy as jnp
from jax import lax
from jax.experimental import pallas as pl
from jax.experimental.pallas import tpu as pltpu
from jax.experimental.pallas import tpu_sc as plsc

N = 10000
E = 320000
D = 128
H = 128
C = 64

NC = 2
NS = 16
NW = NC * NS

CHUNK = 128
EPW = 10240
CPW = EPW // CHUNK
E_PAD = NW * EPW
NP = 10112
RPS = NP // NS
DUMMY = N


@functools.lru_cache(maxsize=None)
def _make_sc_segsum(F):
    mesh = plsc.VectorSubcoreMesh(core_axis_name="c", subcore_axis_name="s")

    @functools.partial(
        pl.kernel,
        out_type=jax.ShapeDtypeStruct((NC, NP, F), jnp.float32),
        mesh=mesh,
        scratch_types=[
            pltpu.VMEM((CPW, CHUNK), jnp.int32),
            pltpu.VMEM((CPW, CHUNK), jnp.int32),
            pltpu.VMEM((CHUNK, F), jnp.float32),
            pltpu.VMEM_SHARED((NP, F), jnp.float32),
            pltpu.SemaphoreType.DMA,
        ],
    )
    def segsum(y_hbm, src_hbm, dst_hbm, zeros_hbm, out_hbm,
               srcb, dstb, rows, acc, sem):
        cid = lax.axis_index("c")
        sid = lax.axis_index("s")
        wid = sid * NC + cid
        base = sid * RPS
        pltpu.sync_copy(zeros_hbm, acc.at[pl.ds(base, RPS)])
        pltpu.sync_copy(src_hbm.at[wid], srcb)
        pltpu.sync_copy(dst_hbm.at[wid], dstb)
        plsc.subcore_barrier()

        @pl.loop(0, CPW)
        def _(j):
            pltpu.async_copy(y_hbm.at[srcb.at[j]], rows, sem).wait()
            pltpu.sync_copy(rows, acc.at[dstb.at[j]], add=True)

        plsc.subcore_barrier()
        pltpu.sync_copy(acc.at[pl.ds(base, RPS)],
                        out_hbm.at[cid].at[pl.ds(base, RPS)])

    return segsum


def _dot_t(x, w):
    return lax.dot_general(x, w, (((1,), (1,)), ((), ())),
                           preferred_element_type=jnp.float32,
                           precision=lax.Precision.HIGHEST)


BLK = 1000


def _l1_body(g_ref, wrel_ref, wroot_ref, b_ref, t_ref, r_ref):
    x = g_ref[...]
    t_ref[...] = _dot_t(x, wrel_ref[...])
    r_ref[...] = _dot_t(x, wroot_ref[...]) + b_ref[...]


def _tc_layer1(g, W1_rel, W1_root, b1):
    return pl.pallas_call(
        _l1_body,
        grid=(N // BLK,),
        in_specs=[
            pl.BlockSpec((BLK, D), lambda i: (i, 0)),
            pl.BlockSpec((H, D), lambda i: (0, 0)),
            pl.BlockSpec((H, D), lambda i: (0, 0)),
            pl.BlockSpec((1, H), lambda i: (0, 0)),
        ],
        out_specs=[
            pl.BlockSpec((BLK, H), lambda i: (i, 0)),
            pl.BlockSpec((BLK, H), lambda i: (i, 0)),
        ],
        out_shape=[jax.ShapeDtypeStruct((N, H), jnp.float32)] * 2,
    )(g, W1_rel, W1_root, b1.reshape(1, H))


def _l2_body(p_ref, r_ref, h_ref):
    h_ref[...] = jnp.maximum(p_ref[0] + p_ref[1] + r_ref[...], 0.0)


def _tc_layer2(p1, r1):
    return pl.pallas_call(
        _l2_body,
        grid=(N // BLK,),
        in_specs=[
            pl.BlockSpec((NC, BLK, H), lambda i: (0, i, 0)),
            pl.BlockSpec((BLK, H), lambda i: (i, 0)),
        ],
        out_specs=pl.BlockSpec((BLK, H), lambda i: (i, 0)),
        out_shape=jax.ShapeDtypeStruct((N, H), jnp.float32),
    )(p1, r1)


def _final_body(p_ref, h_ref, wrel_ref, wroot_ref, b_ref, o_ref):
    agg = p_ref[0] + p_ref[1]
    z = (_dot_t(agg, wrel_ref[...]) + _dot_t(h_ref[...], wroot_ref[...])
         + b_ref[...])
    m = jnp.max(z, axis=-1, keepdims=True)
    lse = jnp.log(jnp.sum(jnp.exp(z - m), axis=-1, keepdims=True)) + m
    o_ref[...] = z - lse


def _tc_final(p2, h, W2_rel, W2_root, b2):
    return pl.pallas_call(
        _final_body,
        grid=(N // BLK,),
        in_specs=[
            pl.BlockSpec((NC, BLK, H), lambda i: (0, i, 0)),
            pl.BlockSpec((BLK, H), lambda i: (i, 0)),
            pl.BlockSpec((C, H), lambda i: (0, 0)),
            pl.BlockSpec((C, H), lambda i: (0, 0)),
            pl.BlockSpec((1, C), lambda i: (0, 0)),
        ],
        out_specs=pl.BlockSpec((BLK, C), lambda i: (i, 0)),
        out_shape=jax.ShapeDtypeStruct((N, C), jnp.float32),
    )(p2, h, W2_rel, W2_root, b2.reshape(1, C))


def kernel(g, in_feat, W1_rel, W1_root, b1, W2_rel, W2_root, b2):
    src = in_feat[0]
    dst = in_feat[1]
    pad = E_PAD - E
    src_p = jnp.concatenate(
        [src, jnp.zeros((pad,), jnp.int32)]).reshape(NW, CPW, CHUNK)
    dst_p = jnp.concatenate(
        [dst, jnp.full((pad,), DUMMY, jnp.int32)]).reshape(NW, CPW, CHUNK)
    zeros_h = jnp.zeros((RPS, H), jnp.float32)

    t1, r1 = _tc_layer1(g, W1_rel, W1_root, b1)
    p1 = _make_sc_segsum(H)(t1, src_p, dst_p, zeros_h)
    h = _tc_layer2(p1, r1)
    p2 = _make_sc_segsum(H)(h, src_p, dst_p, zeros_h)
    return _tc_final(p2, h, W2_rel, W2_root, b2)


# speedup vs baseline: 2.9377x; 1.0695x over previous
import functools

import jax
import jax.numpy as jnp
from jax import lax
from jax.experimental import pallas as pl
from jax.experimental.pallas import tpu as pltpu
from jax.experimental.pallas import tpu_sc as plsc

N = 10000
E = 320000
D = 128
H = 128
C = 64

NC = 2
NS = 16
NW = NC * NS

CHUNK = 128
EPW = 10240
CPW = EPW // CHUNK
PHASES = 2
HALF = CPW // PHASES
E_PAD = NW * EPW
NP = 10112
RPS = NP // NS
DUMMY = N


@functools.lru_cache(maxsize=None)
def _make_sc_segsum(F):
    mesh = plsc.VectorSubcoreMesh(core_axis_name="c", subcore_axis_name="s")

    @functools.partial(
        pl.kernel,
        out_type=jax.ShapeDtypeStruct((NC, NP, F), jnp.float32),
        mesh=mesh,
        scratch_types=[
            pltpu.VMEM((HALF, CHUNK), jnp.int32),
            pltpu.VMEM((HALF, CHUNK), jnp.int32),
            pltpu.VMEM((CHUNK, F), jnp.float32),
            pltpu.VMEM((CHUNK, F), jnp.float32),
            pltpu.VMEM_SHARED((NP, F), jnp.float32),
            pltpu.SemaphoreType.DMA,
            pltpu.SemaphoreType.DMA,
        ],
    )
    def segsum(y_hbm, src_hbm, dst_hbm, zeros_hbm, out_hbm,
               srcb, dstb, rows_a, rows_b, acc, sem_a, sem_b):
        cid = lax.axis_index("c")
        sid = lax.axis_index("s")
        wid = sid * NC + cid
        base = sid * RPS
        pltpu.sync_copy(zeros_hbm, acc.at[pl.ds(base, RPS)])
        plsc.subcore_barrier()

        def gather(i, buf, sem):
            return pltpu.make_async_copy(y_hbm.at[srcb.at[i]], buf, sem)

        def scat(i, buf):
            pltpu.sync_copy(buf, acc.at[dstb.at[i]], add=True)

        @pl.loop(0, PHASES)
        def _(ph):
            pltpu.sync_copy(src_hbm.at[wid].at[pl.ds(ph * HALF, HALF)], srcb)
            pltpu.sync_copy(dst_hbm.at[wid].at[pl.ds(ph * HALF, HALF)], dstb)
            gather(0, rows_a, sem_a).start()

            @pl.loop(0, HALF, step=2)
            def _(j):
                gather(j, rows_a, sem_a).wait()
                gather(j + 1, rows_b, sem_b).start()
                scat(j, rows_a)
                gather(j + 1, rows_b, sem_b).wait()

                @pl.when(j + 2 < HALF)
                def _():
                    gather(j + 2, rows_a, sem_a).start()

                scat(j + 1, rows_b)

        plsc.subcore_barrier()
        pltpu.sync_copy(acc.at[pl.ds(base, RPS)],
                        out_hbm.at[cid].at[pl.ds(base, RPS)])

    return segsum


def _dot_t(x, w):
    return lax.dot_general(x, w, (((1,), (1,)), ((), ())),
                           preferred_element_type=jnp.float32,
                           precision=lax.Precision.HIGHEST)


BLK = 1000


def _l1_body(g_ref, wrel_ref, wroot_ref, b_ref, t_ref, r_ref):
    x = g_ref[...]
    t_ref[...] = _dot_t(x, wrel_ref[...])
    r_ref[...] = _dot_t(x, wroot_ref[...]) + b_ref[...]


def _tc_layer1(g, W1_rel, W1_root, b1):
    return pl.pallas_call(
        _l1_body,
        grid=(N // BLK,),
        in_specs=[
            pl.BlockSpec((BLK, D), lambda i: (i, 0)),
            pl.BlockSpec((H, D), lambda i: (0, 0)),
            pl.BlockSpec((H, D), lambda i: (0, 0)),
            pl.BlockSpec((1, H), lambda i: (0, 0)),
        ],
        out_specs=[
            pl.BlockSpec((BLK, H), lambda i: (i, 0)),
            pl.BlockSpec((BLK, H), lambda i: (i, 0)),
        ],
        out_shape=[jax.ShapeDtypeStruct((N, H), jnp.float32)] * 2,
    )(g, W1_rel, W1_root, b1.reshape(1, H))


def _l2_body(p_ref, r_ref, h_ref):
    h_ref[...] = jnp.maximum(p_ref[0] + p_ref[1] + r_ref[...], 0.0)


def _tc_layer2(p1, r1):
    return pl.pallas_call(
        _l2_body,
        grid=(N // BLK,),
        in_specs=[
            pl.BlockSpec((NC, BLK, H), lambda i: (0, i, 0)),
            pl.BlockSpec((BLK, H), lambda i: (i, 0)),
        ],
        out_specs=pl.BlockSpec((BLK, H), lambda i: (i, 0)),
        out_shape=jax.ShapeDtypeStruct((N, H), jnp.float32),
    )(p1, r1)


def _final_body(p_ref, h_ref, wrel_ref, wroot_ref, b_ref, o_ref):
    agg = p_ref[0] + p_ref[1]
    z = (_dot_t(agg, wrel_ref[...]) + _dot_t(h_ref[...], wroot_ref[...])
         + b_ref[...])
    m = jnp.max(z, axis=-1, keepdims=True)
    lse = jnp.log(jnp.sum(jnp.exp(z - m), axis=-1, keepdims=True)) + m
    o_ref[...] = z - lse


def _tc_final(p2, h, W2_rel, W2_root, b2):
    return pl.pallas_call(
        _final_body,
        grid=(N // BLK,),
        in_specs=[
            pl.BlockSpec((NC, BLK, H), lambda i: (0, i, 0)),
            pl.BlockSpec((BLK, H), lambda i: (i, 0)),
            pl.BlockSpec((C, H), lambda i: (0, 0)),
            pl.BlockSpec((C, H), lambda i: (0, 0)),
            pl.BlockSpec((1, C), lambda i: (0, 0)),
        ],
        out_specs=pl.BlockSpec((BLK, C), lambda i: (i, 0)),
        out_shape=jax.ShapeDtypeStruct((N, C), jnp.float32),
    )(p2, h, W2_rel, W2_root, b2.reshape(1, C))


def kernel(g, in_feat, W1_rel, W1_root, b1, W2_rel, W2_root, b2):
    src = in_feat[0]
    dst = in_feat[1]
    pad = E_PAD - E
    src_p = jnp.concatenate(
        [src, jnp.zeros((pad,), jnp.int32)]).reshape(NW, CPW, CHUNK)
    dst_p = jnp.concatenate(
        [dst, jnp.full((pad,), DUMMY, jnp.int32)]).reshape(NW, CPW, CHUNK)
    zeros_h = jnp.zeros((RPS, H), jnp.float32)

    t1, r1 = _tc_layer1(g, W1_rel, W1_root, b1)
    p1 = _make_sc_segsum(H)(t1, src_p, dst_p, zeros_h)
    h = _tc_layer2(p1, r1)
    p2 = _make_sc_segsum(H)(h, src_p, dst_p, zeros_h)
    return _tc_final(p2, h, W2_rel, W2_root, b2)
